# baseline (speedup 1.0000x reference)
import functools

import jax
import jax.numpy as jnp
from jax import lax
from jax.experimental import pallas as pl
from jax.experimental.pallas import tpu as pltpu
from jax.experimental.pallas import tpu_sc as plsc

N_NODES = 10000
NPAD = 10240
STRIPE = NPAD // 16
N_EDGES = 320000
EPAD = 327680
TRASH = N_NODES

D_IN = 128
DIM = 128
ROWB = 1280
GRID_R = NPAD // ROWB

KD = EPAD // 32 // 128

N_CHUNKS = EPAD // 128

_mesh = plsc.VectorSubcoreMesh(core_axis_name="c", subcore_axis_name="s")


@functools.partial(
    pl.kernel,
    out_type=jax.ShapeDtypeStruct((2 * NPAD,), jnp.float32),
    mesh=_mesh,
    scratch_types=[
        pltpu.VMEM((KD, 128), jnp.int32),
        pltpu.VMEM((128,), jnp.float32),
        pltpu.VMEM_SHARED((NPAD,), jnp.float32),
    ],
)
def _deg_kernel(zeros_hbm, dst_hbm, out_hbm, dst_v, ones_v, acc):
    c = lax.axis_index("c")
    t = lax.axis_index("s")
    pltpu.sync_copy(zeros_hbm, acc.at[pl.ds(t * STRIPE, STRIPE)])
    pltpu.sync_copy(dst_hbm.at[pl.ds((c * 16 + t) * KD, KD)], dst_v)
    for j in range(8):
        ones_v[pl.ds(j * 16, 16)] = jnp.ones((16,), jnp.float32)
    plsc.subcore_barrier()

    def body(i, carry):
        pltpu.sync_copy(ones_v, acc.at[dst_v.at[i]], add=True)
        return carry

    lax.fori_loop(0, KD, body, 0)
    plsc.subcore_barrier()
    pltpu.sync_copy(acc.at[pl.ds(t * STRIPE, STRIPE)],
                    out_hbm.at[pl.ds(c * NPAD + t * STRIPE, STRIPE)])


SUP = 16


def _make_scatter(kf, row_offset):
    assert kf % (2 * SUP) == 0
    n_iter = kf // (2 * SUP)

    @functools.partial(
        pl.kernel,
        out_type=jax.ShapeDtypeStruct((NPAD, 128), jnp.float32),
        mesh=_mesh,
        scratch_types=[
            pltpu.VMEM((2, SUP, 128), jnp.int32),
            pltpu.VMEM((2, SUP, 128), jnp.int32),
            pltpu.VMEM((2, 128, 128), jnp.float32),
            pltpu.VMEM_SHARED((NPAD, 128), jnp.float32),
            pltpu.SemaphoreType.DMA,
            pltpu.SemaphoreType.DMA,
            pltpu.SemaphoreType.DMA,
            pltpu.SemaphoreType.DMA,
            pltpu.SemaphoreType.DMA,
            pltpu.SemaphoreType.DMA,
        ],
    )
    def scat(init_hbm, src_hbm, dst_hbm, g_hbm, out_hbm,
             src_v, dst_v, rows, acc, semg0, semg1, semi0, semi1, sems0, sems1):
        c = lax.axis_index("c")
        t = lax.axis_index("s")
        semg = (semg0, semg1)
        semi = (semi0, semi1)
        sems = (sems0, sems1)
        base = row_offset + t * kf

        def idx_copies(s, b):
            return (
                pltpu.make_async_copy(
                    src_hbm.at[pl.ds(base + s * SUP, SUP)], src_v.at[b], semi[b]),
                pltpu.make_async_copy(
                    dst_hbm.at[pl.ds(base + s * SUP, SUP)], dst_v.at[b], semi[b]),
            )

        def idx_start(s, b):
            for cp in idx_copies(s, b):
                cp.start()

        def idx_wait(s, b):
            for cp in idx_copies(s, b):
                cp.wait()

        def g_copy(j):
            jj = j % (2 * SUP)
            sref = src_v.at[(jj // SUP) % 2].at[jj % SUP]
            return pltpu.make_async_copy(g_hbm.at[sref], rows.at[jj % 2],
                                         semg[jj % 2])

        def s_copy(j):
            dref = dst_v.at[(j // SUP) % 2].at[j % SUP]
            return pltpu.make_async_copy(rows.at[j % 2], acc.at[dref],
                                         sems[j % 2])

        @pl.when(c == 0)
        def _():
            pltpu.sync_copy(init_hbm.at[pl.ds(t * STRIPE, STRIPE)],
                            acc.at[pl.ds(t * STRIPE, STRIPE)])
            plsc.subcore_barrier()
            idx_start(0, 0)
            idx_wait(0, 0)
            idx_start(1, 1)
            g_copy(0).start()

            def block(i, carry):
                s_waited = set()

                def s_wait(j):
                    if j >= 0 and j not in s_waited:
                        s_waited.add(j)
                        s_copy(j).wait()

                for j in range(2 * SUP):
                    if j + 1 < 2 * SUP:
                        if j + 1 == SUP:
                            idx_wait(2 * i + 1, 1)
                        s_wait(j - 1)
                        g_copy(j + 1).start()
                    g_copy(j).wait()
                    s_copy(j).start(add=True)
                    if j == SUP - 1:
                        s_wait(j - 1)
                        s_wait(j)

                        @pl.when(i + 1 < n_iter)
                        def _():
                            idx_start(2 * i + 2, 0)
                s_wait(2 * SUP - 2)
                s_wait(2 * SUP - 1)

                @pl.when(i + 1 < n_iter)
                def _():
                    idx_wait(2 * i + 2, 0)
                    g_copy(0).start()
                    idx_start(2 * i + 3, 1)
                return carry

            lax.fori_loop(0, n_iter, block, 0)
            plsc.subcore_barrier()
            pltpu.sync_copy(acc.at[pl.ds(t * STRIPE, STRIPE)],
                            out_hbm.at[pl.ds(t * STRIPE, STRIPE)])

    return scat


_scatter_a = _make_scatter(128, 0)
_scatter_b = _make_scatter(32, 16 * 128)


def _scatter(zeros_full, src_2, dst_2, g):
    part = _scatter_a(zeros_full, src_2, dst_2, g)
    return _scatter_b(part, src_2, dst_2, g)


def _dis_from(deg_ref):
    deg = deg_ref[0:1, :] + deg_ref[1:2, :] + 1.0
    return lax.rsqrt(deg).reshape(ROWB)


def _tc1_body(deg_ref, x_ref, out_ref):
    dis = _dis_from(deg_ref)
    out_ref[...] = x_ref[...] * dis[:, None]


def _tc2_body(deg_ref, tmp_ref, gx_ref, w1_ref, b1_ref, w3_ref, out_ref):
    dis = _dis_from(deg_ref)
    y = (tmp_ref[...] + gx_ref[...]) * dis[:, None]
    h = jnp.maximum(
        jnp.dot(y, w1_ref[...], preferred_element_type=jnp.float32) + b1_ref[...],
        0.0)
    g2 = jnp.dot(h, w3_ref[...], preferred_element_type=jnp.float32)
    out_ref[...] = g2 * dis[:, None]


def _tc3_body(deg_ref, tmp_ref, g2_ref, b3_ref, out_ref):
    dis = _dis_from(deg_ref)
    out_ref[...] = (tmp_ref[...] + g2_ref[...]) * dis[:, None] + b3_ref[...]


_DEG_SPEC = pl.BlockSpec((2, ROWB), lambda i, *_: (0, i))
_DEG_SPEC1 = pl.BlockSpec((2, ROWB), lambda i: (0, i))


def _tc1(deg2, x_p):
    return pl.pallas_call(
        _tc1_body,
        grid=(GRID_R,),
        in_specs=[
            _DEG_SPEC1,
            pl.BlockSpec((ROWB, 128), lambda i: (i, 0)),
        ],
        out_specs=pl.BlockSpec((ROWB, 128), lambda i: (i, 0)),
        out_shape=jax.ShapeDtypeStruct((NPAD, 128), jnp.float32),
    )(deg2, x_p)


def _tc2(deg2, tmp_x, g_x, W1, b1_2, W3):
    return pl.pallas_call(
        _tc2_body,
        grid=(GRID_R,),
        in_specs=[
            _DEG_SPEC1,
            pl.BlockSpec((ROWB, 128), lambda i: (i, 0)),
            pl.BlockSpec((ROWB, 128), lambda i: (i, 0)),
            pl.BlockSpec((128, 256), lambda i: (0, 0)),
            pl.BlockSpec((1, 256), lambda i: (0, 0)),
            pl.BlockSpec((256, 128), lambda i: (0, 0)),
        ],
        out_specs=pl.BlockSpec((ROWB, 128), lambda i: (i, 0)),
        out_shape=jax.ShapeDtypeStruct((NPAD, 128), jnp.float32),
    )(deg2, tmp_x, g_x, W1, b1_2, W3)


def _tc3(deg2, tmp2, g2, b3_2):
    return pl.pallas_call(
        _tc3_body,
        grid=(GRID_R,),
        in_specs=[
            _DEG_SPEC1,
            pl.BlockSpec((ROWB, 128), lambda i: (i, 0)),
            pl.BlockSpec((ROWB, 128), lambda i: (i, 0)),
            pl.BlockSpec((1, 128), lambda i: (0, 0)),
        ],
        out_specs=pl.BlockSpec((ROWB, 128), lambda i: (i, 0)),
        out_shape=jax.ShapeDtypeStruct((NPAD, 128), jnp.float32),
    )(deg2, tmp2, g2, b3_2)


def kernel(x, edge_index, W1, b1, W3, b3):
    src = edge_index[0].astype(jnp.int32)
    dst = edge_index[1].astype(jnp.int32)
    pad = EPAD - N_EDGES
    src_p = jnp.concatenate([src, jnp.zeros((pad,), jnp.int32)])
    dst_p = jnp.concatenate([dst, jnp.full((pad,), TRASH, jnp.int32)])
    src_2 = src_p.reshape(EPAD // 128, 128)
    dst_2 = dst_p.reshape(EPAD // 128, 128)
    x_p = jnp.pad(x, ((0, NPAD - N_NODES), (0, 0)))
    b1_2 = b1.reshape(1, 256)
    b3_2 = b3.reshape(1, 128)
    zeros1 = jnp.zeros((STRIPE,), jnp.float32)
    zeros2 = jnp.zeros((NPAD, 128), jnp.float32)

    deg_parts = _deg_kernel(zeros1, dst_2)
    deg2 = deg_parts.reshape(2, NPAD)

    g_x = _tc1(deg2, x_p)
    tmp_x = _scatter(zeros2, src_2, dst_2, g_x)
    g2 = _tc2(deg2, tmp_x, g_x, W1, b1_2, W3)
    tmp2 = _scatter(zeros2, src_2, dst_2, g2)
    out = _tc3(deg2, tmp2, g2, b3_2)
    return out[:N_NODES]


# speedup vs baseline: 1.3066x; 1.3066x over previous
import functools

import jax
import jax.numpy as jnp
from jax import lax
from jax.experimental import pallas as pl
from jax.experimental.pallas import tpu as pltpu
from jax.experimental.pallas import tpu_sc as plsc

N_NODES = 10000
NPAD = 10240
STRIPE = NPAD // 16
N_EDGES = 320000
EPAD = 327680
TRASH = N_NODES

D_IN = 128
DIM = 128
ROWB = 1280
GRID_R = NPAD // ROWB

KD = EPAD // 32 // 128

N_CHUNKS = EPAD // 128

_mesh = plsc.VectorSubcoreMesh(core_axis_name="c", subcore_axis_name="s")


@functools.partial(
    pl.kernel,
    out_type=jax.ShapeDtypeStruct((2 * NPAD,), jnp.float32),
    mesh=_mesh,
    scratch_types=[
        pltpu.VMEM((KD, 128), jnp.int32),
        pltpu.VMEM((128,), jnp.float32),
        pltpu.VMEM_SHARED((NPAD,), jnp.float32),
    ],
)
def _deg_kernel(zeros_hbm, dst_hbm, out_hbm, dst_v, ones_v, acc):
    c = lax.axis_index("c")
    t = lax.axis_index("s")
    pltpu.sync_copy(zeros_hbm, acc.at[pl.ds(t * STRIPE, STRIPE)])
    pltpu.sync_copy(dst_hbm.at[pl.ds((c * 16 + t) * KD, KD)], dst_v)
    for j in range(8):
        ones_v[pl.ds(j * 16, 16)] = jnp.ones((16,), jnp.float32)
    plsc.subcore_barrier()

    def body(i, carry):
        pltpu.sync_copy(ones_v, acc.at[dst_v.at[i]], add=True)
        return carry

    lax.fori_loop(0, KD, body, 0)
    plsc.subcore_barrier()
    pltpu.sync_copy(acc.at[pl.ds(t * STRIPE, STRIPE)],
                    out_hbm.at[pl.ds(c * NPAD + t * STRIPE, STRIPE)])


SUP = 16


def _make_scatter():
    k_chunks = N_CHUNKS // 32
    assert k_chunks % SUP == 0
    n_sup = k_chunks // SUP

    @functools.partial(
        pl.kernel,
        out_type=jax.ShapeDtypeStruct((2 * NPAD, 128), jnp.float32),
        mesh=_mesh,
        scratch_types=[
            pltpu.VMEM((2, SUP, 128), jnp.int32),
            pltpu.VMEM((2, SUP, 128), jnp.int32),
            pltpu.VMEM((2, 128, 128), jnp.float32),
            pltpu.VMEM_SHARED((NPAD, 128), jnp.float32),
            pltpu.SemaphoreType.DMA,
            pltpu.SemaphoreType.DMA,
            pltpu.SemaphoreType.DMA,
            pltpu.SemaphoreType.DMA,
            pltpu.SemaphoreType.DMA,
            pltpu.SemaphoreType.DMA,
        ],
    )
    def scat(zeros_hbm, src_hbm, dst_hbm, g_hbm, out_hbm,
             src_v, dst_v, rows, acc, semg0, semg1, semi0, semi1, sems0, sems1):
        c = lax.axis_index("c")
        t = lax.axis_index("s")
        semg = (semg0, semg1)
        semi = (semi0, semi1)
        sems = (sems0, sems1)
        base = c * (N_CHUNKS // 2) + t * k_chunks

        def idx_copies(s):
            b = s % 2
            return (
                pltpu.make_async_copy(
                    src_hbm.at[pl.ds(base + s * SUP, SUP)], src_v.at[b], semi[b]),
                pltpu.make_async_copy(
                    dst_hbm.at[pl.ds(base + s * SUP, SUP)], dst_v.at[b], semi[b]),
            )

        def idx_start(s):
            for cp in idx_copies(s):
                cp.start()

        def idx_wait(s):
            for cp in idx_copies(s):
                cp.wait()

        def g_copy(ci):
            sref = src_v.at[(ci // SUP) % 2].at[ci % SUP]
            return pltpu.make_async_copy(g_hbm.at[sref], rows.at[ci % 2],
                                         semg[ci % 2])

        def s_copy(ci):
            dref = dst_v.at[(ci // SUP) % 2].at[ci % SUP]
            return pltpu.make_async_copy(rows.at[ci % 2], acc.at[dref],
                                         sems[ci % 2])

        s_waited = set()

        def s_wait(i):
            if 0 <= i < k_chunks and i not in s_waited:
                s_waited.add(i)
                s_copy(i).wait()

        pltpu.sync_copy(zeros_hbm, acc.at[pl.ds(t * STRIPE, STRIPE)])
        plsc.subcore_barrier()
        idx_start(0)
        idx_wait(0)
        idx_start(1)
        g_copy(0).start()
        for ci in range(k_chunks):
            nxt = ci + 1
            if nxt < k_chunks:
                if nxt % SUP == 0:
                    idx_wait(nxt // SUP)
                s_wait(nxt - 2)
                g_copy(nxt).start()
            g_copy(ci).wait()
            s_copy(ci).start(add=True)
            if nxt % SUP == 0 and nxt // SUP + 1 < n_sup:
                s_wait(ci - 1)
                s_wait(ci)
                idx_start(nxt // SUP + 1)
        s_wait(k_chunks - 2)
        s_wait(k_chunks - 1)

        plsc.subcore_barrier()
        pltpu.sync_copy(acc.at[pl.ds(t * STRIPE, STRIPE)],
                        out_hbm.at[pl.ds(c * NPAD + t * STRIPE, STRIPE)])

    return scat


_scatter = _make_scatter()


def _dis_from(deg_ref):
    deg = deg_ref[0:1, :] + deg_ref[1:2, :] + 1.0
    return lax.rsqrt(deg).reshape(ROWB)


def _tc1_body(deg_ref, x_ref, out_ref):
    dis = _dis_from(deg_ref)
    out_ref[...] = x_ref[...] * dis[:, None]


def _tc2_body(deg_ref, tmp_ref, gx_ref, w1_ref, b1_ref, w3_ref, out_ref):
    dis = _dis_from(deg_ref)
    y = (tmp_ref[0] + tmp_ref[1] + gx_ref[...]) * dis[:, None]
    h = jnp.maximum(
        jnp.dot(y, w1_ref[...], preferred_element_type=jnp.float32) + b1_ref[...],
        0.0)
    g2 = jnp.dot(h, w3_ref[...], preferred_element_type=jnp.float32)
    out_ref[...] = g2 * dis[:, None]


def _tc3_body(deg_ref, tmp_ref, g2_ref, b3_ref, out_ref):
    dis = _dis_from(deg_ref)
    out_ref[...] = (tmp_ref[0] + tmp_ref[1] + g2_ref[...]) * dis[:, None] + b3_ref[...]


_DEG_SPEC = pl.BlockSpec((2, ROWB), lambda i, *_: (0, i))
_DEG_SPEC1 = pl.BlockSpec((2, ROWB), lambda i: (0, i))


def _tc1(deg2, x_p):
    return pl.pallas_call(
        _tc1_body,
        grid=(GRID_R,),
        in_specs=[
            _DEG_SPEC1,
            pl.BlockSpec((ROWB, 128), lambda i: (i, 0)),
        ],
        out_specs=pl.BlockSpec((ROWB, 128), lambda i: (i, 0)),
        out_shape=jax.ShapeDtypeStruct((NPAD, 128), jnp.float32),
    )(deg2, x_p)


def _tc2(deg2, tmp_x, g_x, W1, b1_2, W3):
    return pl.pallas_call(
        _tc2_body,
        grid=(GRID_R,),
        in_specs=[
            _DEG_SPEC1,
            pl.BlockSpec((2, ROWB, 128), lambda i: (0, i, 0)),
            pl.BlockSpec((ROWB, 128), lambda i: (i, 0)),
            pl.BlockSpec((128, 256), lambda i: (0, 0)),
            pl.BlockSpec((1, 256), lambda i: (0, 0)),
            pl.BlockSpec((256, 128), lambda i: (0, 0)),
        ],
        out_specs=pl.BlockSpec((ROWB, 128), lambda i: (i, 0)),
        out_shape=jax.ShapeDtypeStruct((NPAD, 128), jnp.float32),
    )(deg2, tmp_x, g_x, W1, b1_2, W3)


def _tc3(deg2, tmp2, g2, b3_2):
    return pl.pallas_call(
        _tc3_body,
        grid=(GRID_R,),
        in_specs=[
            _DEG_SPEC1,
            pl.BlockSpec((2, ROWB, 128), lambda i: (0, i, 0)),
            pl.BlockSpec((ROWB, 128), lambda i: (i, 0)),
            pl.BlockSpec((1, 128), lambda i: (0, 0)),
        ],
        out_specs=pl.BlockSpec((ROWB, 128), lambda i: (i, 0)),
        out_shape=jax.ShapeDtypeStruct((NPAD, 128), jnp.float32),
    )(deg2, tmp2, g2, b3_2)


def kernel(x, edge_index, W1, b1, W3, b3):
    src = edge_index[0].astype(jnp.int32)
    dst = edge_index[1].astype(jnp.int32)
    pad = EPAD - N_EDGES
    src_p = jnp.concatenate([src, jnp.zeros((pad,), jnp.int32)])
    trash_dst = TRASH + (jnp.arange(pad, dtype=jnp.int32) % (NPAD - TRASH))
    dst_p = jnp.concatenate([dst, trash_dst])
    src_2 = src_p.reshape(EPAD // 128, 128)
    dst_2 = dst_p.reshape(EPAD // 128, 128)
    x_p = jnp.pad(x, ((0, NPAD - N_NODES), (0, 0)))
    b1_2 = b1.reshape(1, 256)
    b3_2 = b3.reshape(1, 128)
    zeros1 = jnp.zeros((STRIPE,), jnp.float32)
    zeros2 = jnp.zeros((STRIPE, 128), jnp.float32)

    deg_parts = _deg_kernel(zeros1, dst_2)
    deg2 = deg_parts.reshape(2, NPAD)

    g_x = _tc1(deg2, x_p)
    tmp_x = _scatter(zeros2, src_2, dst_2, g_x)
    g2 = _tc2(deg2, tmp_x.reshape(2, NPAD, 128), g_x, W1, b1_2, W3)
    tmp2 = _scatter(zeros2, src_2, dst_2, g2)
    out = _tc3(deg2, tmp2.reshape(2, NPAD, 128), g2, b3_2)
    return out[:N_NODES]


# speedup vs baseline: 4.0300x; 3.0843x over previous
import functools

import jax
import jax.numpy as jnp
from jax import lax
from jax.experimental import pallas as pl
from jax.experimental.pallas import tpu as pltpu
from jax.experimental.pallas import tpu_sc as plsc

N_NODES = 10000
NPAD = 10240
STRIPE = NPAD // 16
N_EDGES = 320000
EPAD = 327680
TRASH = N_NODES

D_IN = 128
DIM = 128
ROWB = 1280
GRID_R = NPAD // ROWB

KD = EPAD // 32 // 128

N_CHUNKS = EPAD // 128

_mesh = plsc.VectorSubcoreMesh(core_axis_name="c", subcore_axis_name="s")


@functools.partial(
    pl.kernel,
    out_type=jax.ShapeDtypeStruct((2 * NPAD,), jnp.float32),
    mesh=_mesh,
    scratch_types=[
        pltpu.VMEM((KD, 128), jnp.int32),
        pltpu.VMEM((128,), jnp.float32),
        pltpu.VMEM_SHARED((NPAD,), jnp.float32),
    ],
)
def _deg_kernel(zeros_hbm, dst_hbm, out_hbm, dst_v, ones_v, acc):
    c = lax.axis_index("c")
    t = lax.axis_index("s")
    pltpu.sync_copy(zeros_hbm, acc.at[pl.ds(t * STRIPE, STRIPE)])
    pltpu.sync_copy(dst_hbm.at[pl.ds((c * 16 + t) * KD, KD)], dst_v)
    for j in range(8):
        ones_v[pl.ds(j * 16, 16)] = jnp.ones((16,), jnp.float32)
    plsc.subcore_barrier()

    def body(i, carry):
        pltpu.sync_copy(ones_v, acc.at[dst_v.at[i]], add=True)
        return carry

    lax.fori_loop(0, KD, body, 0)
    plsc.subcore_barrier()
    pltpu.sync_copy(acc.at[pl.ds(t * STRIPE, STRIPE)],
                    out_hbm.at[pl.ds(c * NPAD + t * STRIPE, STRIPE)])


SUP = 16


def _make_scatter():
    k_chunks = N_CHUNKS // 32
    assert k_chunks % SUP == 0
    n_sup = k_chunks // SUP

    @functools.partial(
        pl.kernel,
        out_type=jax.ShapeDtypeStruct((2 * NPAD, 128), jnp.float32),
        mesh=_mesh,
        scratch_types=[
            pltpu.VMEM((2, SUP, 128), jnp.int32),
            pltpu.VMEM((2, SUP, 128), jnp.int32),
            pltpu.VMEM((2, 128, 128), jnp.float32),
            pltpu.VMEM_SHARED((NPAD, 128), jnp.float32),
            pltpu.SemaphoreType.DMA,
            pltpu.SemaphoreType.DMA,
            pltpu.SemaphoreType.DMA,
            pltpu.SemaphoreType.DMA,
            pltpu.SemaphoreType.DMA,
            pltpu.SemaphoreType.DMA,
        ],
    )
    def scat(zeros_hbm, src_hbm, dst_hbm, g_hbm, out_hbm,
             src_v, dst_v, rows, acc, semg0, semg1, semi0, semi1, sems0, sems1):
        c = lax.axis_index("c")
        t = lax.axis_index("s")
        semg = (semg0, semg1)
        semi = (semi0, semi1)
        sems = (sems0, sems1)
        base = c * (N_CHUNKS // 2) + t * k_chunks

        def idx_copies(s):
            b = s % 2
            return (
                pltpu.make_async_copy(
                    src_hbm.at[pl.ds(base + s * SUP, SUP)], src_v.at[b], semi[b]),
                pltpu.make_async_copy(
                    dst_hbm.at[pl.ds(base + s * SUP, SUP)], dst_v.at[b], semi[b]),
            )

        def idx_start(s):
            for cp in idx_copies(s):
                cp.start()

        def idx_wait(s):
            for cp in idx_copies(s):
                cp.wait()

        def g_copy(ci):
            sref = src_v.at[(ci // SUP) % 2].at[ci % SUP]
            return pltpu.make_async_copy(g_hbm.at[sref], rows.at[ci % 2],
                                         semg[ci % 2])

        def s_copy(ci):
            dref = dst_v.at[(ci // SUP) % 2].at[ci % SUP]
            return pltpu.make_async_copy(rows.at[ci % 2], acc.at[dref],
                                         sems[ci % 2])

        s_waited = set()

        def s_wait(i):
            if 0 <= i < k_chunks and i not in s_waited:
                s_waited.add(i)
                s_copy(i).wait()

        pltpu.sync_copy(zeros_hbm, acc.at[pl.ds(t * STRIPE, STRIPE)])
        plsc.subcore_barrier()
        idx_start(0)
        idx_wait(0)
        idx_start(1)
        g_copy(0).start()
        for ci in range(k_chunks):
            nxt = ci + 1
            if nxt < k_chunks:
                if nxt % SUP == 0:
                    idx_wait(nxt // SUP)
                s_wait(nxt - 2)
                g_copy(nxt).start()
            g_copy(ci).wait()
            s_copy(ci).start(add=True)
            if nxt % SUP == 0 and nxt // SUP + 1 < n_sup:
                s_wait(ci - 1)
                s_wait(ci)
                idx_start(nxt // SUP + 1)
        s_wait(k_chunks - 2)
        s_wait(k_chunks - 1)

        plsc.subcore_barrier()
        pltpu.sync_copy(acc.at[pl.ds(t * STRIPE, STRIPE)],
                        out_hbm.at[pl.ds(c * NPAD + t * STRIPE, STRIPE)])

    return scat


_scatter = _make_scatter()


def _dis_from(deg_ref):
    deg = deg_ref[0:1, :] + deg_ref[1:2, :] + 1.0
    return lax.rsqrt(deg).reshape(ROWB)


def _tc1_body(deg_ref, x_ref, out_ref):
    dis = _dis_from(deg_ref)
    out_ref[...] = x_ref[...] * dis[:, None]


def _tc2_body(deg_ref, tmp_ref, gx_ref, w1_ref, b1_ref, w3_ref, out_ref):
    dis = _dis_from(deg_ref)
    y = (tmp_ref[0] + tmp_ref[1] + gx_ref[...]) * dis[:, None]
    h = jnp.maximum(
        jnp.dot(y, w1_ref[...], preferred_element_type=jnp.float32) + b1_ref[...],
        0.0)
    g2 = jnp.dot(h, w3_ref[...], preferred_element_type=jnp.float32)
    row = pl.program_id(0) * ROWB + lax.broadcasted_iota(jnp.int32, (ROWB, 1), 0)
    out_ref[...] = jnp.where(row < N_NODES, g2 * dis[:, None], 0.0)


def _tc3_body(deg_ref, tmp_ref, g2_ref, b3_ref, out_ref):
    dis = _dis_from(deg_ref)
    out_ref[...] = (tmp_ref[0] + tmp_ref[1] + g2_ref[...]) * dis[:, None] + b3_ref[...]


_DEG_SPEC = pl.BlockSpec((2, ROWB), lambda i, *_: (0, i))
_DEG_SPEC1 = pl.BlockSpec((2, ROWB), lambda i: (0, i))


def _tc1(deg2, x_p):
    return pl.pallas_call(
        _tc1_body,
        grid=(GRID_R,),
        in_specs=[
            _DEG_SPEC1,
            pl.BlockSpec((ROWB, 128), lambda i: (i, 0)),
        ],
        out_specs=pl.BlockSpec((ROWB, 128), lambda i: (i, 0)),
        out_shape=jax.ShapeDtypeStruct((NPAD, 128), jnp.float32),
    )(deg2, x_p)


def _tc2(deg2, tmp_x, g_x, W1, b1_2, W3):
    return pl.pallas_call(
        _tc2_body,
        grid=(GRID_R,),
        in_specs=[
            _DEG_SPEC1,
            pl.BlockSpec((2, ROWB, 128), lambda i: (0, i, 0)),
            pl.BlockSpec((ROWB, 128), lambda i: (i, 0)),
            pl.BlockSpec((128, 256), lambda i: (0, 0)),
            pl.BlockSpec((1, 256), lambda i: (0, 0)),
            pl.BlockSpec((256, 128), lambda i: (0, 0)),
        ],
        out_specs=pl.BlockSpec((ROWB, 128), lambda i: (i, 0)),
        out_shape=jax.ShapeDtypeStruct((NPAD, 128), jnp.float32),
    )(deg2, tmp_x, g_x, W1, b1_2, W3)


def _tc3(deg2, tmp2, g2, b3_2):
    return pl.pallas_call(
        _tc3_body,
        grid=(GRID_R,),
        in_specs=[
            _DEG_SPEC1,
            pl.BlockSpec((2, ROWB, 128), lambda i: (0, i, 0)),
            pl.BlockSpec((ROWB, 128), lambda i: (i, 0)),
            pl.BlockSpec((1, 128), lambda i: (0, 0)),
        ],
        out_specs=pl.BlockSpec((ROWB, 128), lambda i: (i, 0)),
        out_shape=jax.ShapeDtypeStruct((NPAD, 128), jnp.float32),
    )(deg2, tmp2, g2, b3_2)


def kernel(x, edge_index, W1, b1, W3, b3):
    src = edge_index[0].astype(jnp.int32)
    dst = edge_index[1].astype(jnp.int32)
    pad = EPAD - N_EDGES
    pad_ar = jnp.arange(pad, dtype=jnp.int32)
    src_p = jnp.concatenate([src, TRASH + pad_ar % (NPAD - TRASH)])
    dst_msg = jnp.concatenate([dst, pad_ar % N_NODES])
    dst_deg = jnp.concatenate([dst, TRASH + pad_ar % (NPAD - TRASH)])
    src_2 = src_p.reshape(EPAD // 128, 128)
    dst_2 = dst_msg.reshape(EPAD // 128, 128)
    dst_2d = dst_deg.reshape(EPAD // 128, 128)
    x_p = jnp.pad(x, ((0, NPAD - N_NODES), (0, 0)))
    b1_2 = b1.reshape(1, 256)
    b3_2 = b3.reshape(1, 128)
    zeros1 = jnp.zeros((STRIPE,), jnp.float32)
    zeros2 = jnp.zeros((STRIPE, 128), jnp.float32)

    deg_parts = _deg_kernel(zeros1, dst_2d)
    deg2 = deg_parts.reshape(2, NPAD)

    g_x = _tc1(deg2, x_p)
    tmp_x = _scatter(zeros2, src_2, dst_2, g_x)
    g2 = _tc2(deg2, tmp_x.reshape(2, NPAD, 128), g_x, W1, b1_2, W3)
    tmp2 = _scatter(zeros2, src_2, dst_2, g2)
    out = _tc3(deg2, tmp2.reshape(2, NPAD, 128), g2, b3_2)
    return out[:N_NODES]


# speedup vs baseline: 4.0337x; 1.0009x over previous
import functools

import jax
import jax.numpy as jnp
from jax import lax
from jax.experimental import pallas as pl
from jax.experimental.pallas import tpu as pltpu
from jax.experimental.pallas import tpu_sc as plsc

N_NODES = 10000
NPAD = 10240
STRIPE = NPAD // 16
N_EDGES = 320000
EPAD = 327680
TRASH = N_NODES

ROWB = 1280
GRID_R = NPAD // ROWB

KD = EPAD // 32 // 128
N_CHUNKS = EPAD // 128

_mesh = plsc.VectorSubcoreMesh(core_axis_name="c", subcore_axis_name="s")


@functools.partial(
    pl.kernel,
    out_type=jax.ShapeDtypeStruct((2 * NPAD,), jnp.float32),
    mesh=_mesh,
    scratch_types=[
        pltpu.VMEM((KD, 128), jnp.int32),
        pltpu.VMEM((128,), jnp.float32),
        pltpu.VMEM_SHARED((NPAD,), jnp.float32),
    ],
)
def _deg_kernel(zeros_hbm, dst_hbm, out_hbm, dst_v, ones_v, acc):
    c = lax.axis_index("c")
    t = lax.axis_index("s")
    pltpu.sync_copy(zeros_hbm, acc.at[pl.ds(t * STRIPE, STRIPE)])
    pltpu.sync_copy(dst_hbm.at[pl.ds((c * 16 + t) * KD, KD)], dst_v)
    for j in range(8):
        ones_v[pl.ds(j * 16, 16)] = jnp.ones((16,), jnp.float32)
    plsc.subcore_barrier()

    def body(i, carry):
        pltpu.sync_copy(ones_v, acc.at[dst_v.at[i]], add=True)
        return carry

    lax.fori_loop(0, KD, body, 0)
    plsc.subcore_barrier()
    pltpu.sync_copy(acc.at[pl.ds(t * STRIPE, STRIPE)],
                    out_hbm.at[pl.ds(c * NPAD + t * STRIPE, STRIPE)])


SUP = 16


def _make_scatter():
    k_chunks = N_CHUNKS // 32
    assert k_chunks % SUP == 0
    n_sup = k_chunks // SUP

    @functools.partial(
        pl.kernel,
        out_type=jax.ShapeDtypeStruct((2 * NPAD, 128), jnp.float32),
        mesh=_mesh,
        scratch_types=[
            pltpu.VMEM((2, SUP, 128), jnp.int32),
            pltpu.VMEM((2, SUP, 128), jnp.int32),
            pltpu.VMEM((2, 128, 128), jnp.float32),
            pltpu.VMEM_SHARED((NPAD, 128), jnp.float32),
            pltpu.SemaphoreType.DMA,
            pltpu.SemaphoreType.DMA,
            pltpu.SemaphoreType.DMA,
            pltpu.SemaphoreType.DMA,
            pltpu.SemaphoreType.DMA,
            pltpu.SemaphoreType.DMA,
        ],
    )
    def scat(zeros_hbm, src_hbm, dst_hbm, g_hbm, out_hbm,
             src_v, dst_v, rows, acc, semg0, semg1, semi0, semi1, sems0, sems1):
        c = lax.axis_index("c")
        t = lax.axis_index("s")
        semg = (semg0, semg1)
        semi = (semi0, semi1)
        sems = (sems0, sems1)
        base = c * (N_CHUNKS // 2) + t * k_chunks

        def idx_copies(s):
            b = s % 2
            return (
                pltpu.make_async_copy(
                    src_hbm.at[pl.ds(base + s * SUP, SUP)], src_v.at[b], semi[b]),
                pltpu.make_async_copy(
                    dst_hbm.at[pl.ds(base + s * SUP, SUP)], dst_v.at[b], semi[b]),
            )

        def idx_start(s):
            for cp in idx_copies(s):
                cp.start()

        def idx_wait(s):
            for cp in idx_copies(s):
                cp.wait()

        def g_copy(ci):
            sref = src_v.at[(ci // SUP) % 2].at[ci % SUP]
            return pltpu.make_async_copy(g_hbm.at[sref], rows.at[ci % 2],
                                         semg[ci % 2])

        def s_copy(ci):
            dref = dst_v.at[(ci // SUP) % 2].at[ci % SUP]
            return pltpu.make_async_copy(rows.at[ci % 2], acc.at[dref],
                                         sems[ci % 2])

        s_waited = set()

        def s_wait(i):
            if 0 <= i < k_chunks and i not in s_waited:
                s_waited.add(i)
                s_copy(i).wait()

        pltpu.sync_copy(zeros_hbm, acc.at[pl.ds(t * STRIPE, STRIPE)])
        plsc.subcore_barrier()
        idx_start(0)
        idx_wait(0)
        idx_start(1)
        g_copy(0).start()
        for ci in range(k_chunks):
            nxt = ci + 1
            if nxt < k_chunks:
                if nxt % SUP == 0:
                    idx_wait(nxt // SUP)
                s_wait(nxt - 2)
                g_copy(nxt).start()
            g_copy(ci).wait()
            s_copy(ci).start(add=True)
            if nxt % SUP == 0 and nxt // SUP + 1 < n_sup:
                s_wait(ci - 1)
                s_wait(ci)
                idx_start(nxt // SUP + 1)
        s_wait(k_chunks - 2)
        s_wait(k_chunks - 1)

        plsc.subcore_barrier()
        pltpu.sync_copy(acc.at[pl.ds(t * STRIPE, STRIPE)],
                        out_hbm.at[pl.ds(c * NPAD + t * STRIPE, STRIPE)])

    return scat


_scatter = _make_scatter()


def _dis_from(deg_ref):
    deg = deg_ref[0:1, :] + deg_ref[1:2, :] + 1.0
    return lax.rsqrt(deg).reshape(ROWB)


def _tc1_body(deg_ref, x_ref, out_ref):
    dis = _dis_from(deg_ref)
    out_ref[...] = x_ref[...] * dis[:, None]


def _tc2_body(deg_ref, tmp_ref, gx_ref, w1_ref, b1_ref, w3_ref, out_ref):
    dis = _dis_from(deg_ref)
    y = (tmp_ref[0] + tmp_ref[1] + gx_ref[...]) * dis[:, None]
    h = jnp.maximum(
        jnp.dot(y, w1_ref[...], preferred_element_type=jnp.float32) + b1_ref[...],
        0.0)
    g2 = jnp.dot(h, w3_ref[...], preferred_element_type=jnp.float32)
    row = pl.program_id(0) * ROWB + lax.broadcasted_iota(jnp.int32, (ROWB, 1), 0)
    out_ref[...] = jnp.where(row < N_NODES, g2 * dis[:, None], 0.0)


def _tc3_body(deg_ref, tmp_ref, g2_ref, b3_ref, out_ref):
    dis = _dis_from(deg_ref)
    out_ref[...] = (tmp_ref[0] + tmp_ref[1] + g2_ref[...]) * dis[:, None] + b3_ref[...]


_DEG_SPEC1 = pl.BlockSpec((2, ROWB), lambda i: (0, i))


def _tc1(deg2, x_p):
    return pl.pallas_call(
        _tc1_body,
        grid=(GRID_R,),
        in_specs=[
            _DEG_SPEC1,
            pl.BlockSpec((ROWB, 128), lambda i: (i, 0)),
        ],
        out_specs=pl.BlockSpec((ROWB, 128), lambda i: (i, 0)),
        out_shape=jax.ShapeDtypeStruct((NPAD, 128), jnp.float32),
    )(deg2, x_p)


def _tc2(deg2, tmp_x, g_x, W1, b1_2, W3):
    return pl.pallas_call(
        _tc2_body,
        grid=(GRID_R,),
        in_specs=[
            _DEG_SPEC1,
            pl.BlockSpec((2, ROWB, 128), lambda i: (0, i, 0)),
            pl.BlockSpec((ROWB, 128), lambda i: (i, 0)),
            pl.BlockSpec((128, 256), lambda i: (0, 0)),
            pl.BlockSpec((1, 256), lambda i: (0, 0)),
            pl.BlockSpec((256, 128), lambda i: (0, 0)),
        ],
        out_specs=pl.BlockSpec((ROWB, 128), lambda i: (i, 0)),
        out_shape=jax.ShapeDtypeStruct((NPAD, 128), jnp.float32),
    )(deg2, tmp_x, g_x, W1, b1_2, W3)


def _tc3(deg2, tmp2, g2, b3_2):
    return pl.pallas_call(
        _tc3_body,
        grid=(GRID_R,),
        in_specs=[
            _DEG_SPEC1,
            pl.BlockSpec((2, ROWB, 128), lambda i: (0, i, 0)),
            pl.BlockSpec((ROWB, 128), lambda i: (i, 0)),
            pl.BlockSpec((1, 128), lambda i: (0, 0)),
        ],
        out_specs=pl.BlockSpec((ROWB, 128), lambda i: (i, 0)),
        out_shape=jax.ShapeDtypeStruct((NPAD, 128), jnp.float32),
    )(deg2, tmp2, g2, b3_2)


def kernel(x, edge_index, W1, b1, W3, b3):
    src = edge_index[0].astype(jnp.int32)
    dst = edge_index[1].astype(jnp.int32)
    pad = EPAD - N_EDGES
    pad_ar = jnp.arange(pad, dtype=jnp.int32)
    src_p = jnp.concatenate([src, TRASH + pad_ar % (NPAD - TRASH)])
    dst_msg = jnp.concatenate([dst, pad_ar % N_NODES])
    dst_deg = jnp.concatenate([dst, TRASH + pad_ar % (NPAD - TRASH)])
    src_2 = src_p.reshape(EPAD // 128, 128)
    dst_2 = dst_msg.reshape(EPAD // 128, 128)
    dst_2d = dst_deg.reshape(EPAD // 128, 128)
    x_p = jnp.pad(x, ((0, NPAD - N_NODES), (0, 0)))
    b1_2 = b1.reshape(1, 256)
    b3_2 = b3.reshape(1, 128)
    zeros1 = jnp.zeros((STRIPE,), jnp.float32)
    zeros2 = jnp.zeros((STRIPE, 128), jnp.float32)

    deg_parts = _deg_kernel(zeros1, dst_2d)
    deg2 = deg_parts.reshape(2, NPAD)

    g_x = _tc1(deg2, x_p)
    tmp_x = _scatter(zeros2, src_2, dst_2, g_x)
    g2 = _tc2(deg2, tmp_x.reshape(2, NPAD, 128), g_x, W1, b1_2, W3)
    tmp2 = _scatter(zeros2, src_2, dst_2, g2)
    out = _tc3(deg2, tmp2.reshape(2, NPAD, 128), g2, b3_2)
    return out[:N_NODES]
